# Initial kernel scaffold; baseline (speedup 1.0000x reference)
import jax
import jax.numpy as jnp
from jax.experimental import pallas as pl


def kernel(x_user, x_movie, edge_index_um, edge_index_mu, edge_label_index, l1_um_Ws, l1_um_Wd, l1_um_as, l1_um_ad, l1_um_b, l2_um_Ws, l2_um_Wd, l2_um_as, l2_um_ad, l2_um_b, l1_mu_Ws, l1_mu_Wd, l1_mu_as, l1_mu_ad, l1_mu_b, l2_mu_Ws, l2_mu_Wd, l2_mu_as, l2_mu_ad, l2_mu_b, dec_W1, dec_b1, dec_W2, dec_b2):
    raise NotImplementedError("write your pallas kernel here")


# speedup vs baseline: 1.0506x; 1.0506x over previous
import jax
import jax.numpy as jnp
from jax.experimental import pallas as pl
from jax.experimental.pallas import tpu as pltpu

H1, C1 = 8, 32
H2, C2 = 4, 256
D1 = H1 * C1
D2 = H2 * C2


def _gat_jnp(x_src, x_dst, row, col, Ws, Wd, a_s, a_d, b, H, C):
    n_dst = x_dst.shape[0]
    hs = (x_src @ Ws).reshape(-1, H, C)
    alpha_s = jnp.sum(hs * a_s[None], axis=-1)
    alpha_d = jnp.sum((x_dst @ Wd).reshape(-1, H, C) * a_d[None], axis=-1)
    e = alpha_s[row] + alpha_d[col]
    e = jnp.where(e >= 0, e, 0.2 * e)
    ex = jnp.exp(e)
    den = jax.ops.segment_sum(ex, col, num_segments=n_dst)
    msg = hs[row] * ex[:, :, None]
    agg = jax.ops.segment_sum(msg, col, num_segments=n_dst)
    out = agg / (den[:, :, None] + 1e-16)
    return out.reshape(n_dst, H * C) + b


def _dec_body(zr_ref, zc_ref, w1a_ref, w1b_ref, b1_ref, w2_ref, b2_ref, o_ref):
    z = zr_ref[...] @ w1a_ref[...] + zc_ref[...] @ w1b_ref[...] + b1_ref[...]
    z = jnp.maximum(z, 0.0)
    o_ref[...] = (z @ w2_ref[...] + b2_ref[...])[:, 0]


def _decoder(zr, zc, W1, b1, W2, b2):
    B = zr.shape[0]
    TB = 2048
    W1a, W1b = W1[:D2], W1[D2:]
    return pl.pallas_call(
        _dec_body,
        grid=(B // TB,),
        in_specs=[
            pl.BlockSpec((TB, D2), lambda i: (i, 0)),
            pl.BlockSpec((TB, D2), lambda i: (i, 0)),
            pl.BlockSpec((D2, 32), lambda i: (0, 0)),
            pl.BlockSpec((D2, 32), lambda i: (0, 0)),
            pl.BlockSpec((32,), lambda i: (0,)),
            pl.BlockSpec((32, 1), lambda i: (0, 0)),
            pl.BlockSpec((1,), lambda i: (0,)),
        ],
        out_specs=pl.BlockSpec((TB,), lambda i: (i,)),
        out_shape=jax.ShapeDtypeStruct((B,), jnp.float32),
    )(zr, zc, W1a, W1b, b1, W2, b2)


def kernel(x_user, x_movie, edge_index_um, edge_index_mu, edge_label_index,
           l1_um_Ws, l1_um_Wd, l1_um_as, l1_um_ad, l1_um_b,
           l2_um_Ws, l2_um_Wd, l2_um_as, l2_um_ad, l2_um_b,
           l1_mu_Ws, l1_mu_Wd, l1_mu_as, l1_mu_ad, l1_mu_b,
           l2_mu_Ws, l2_mu_Wd, l2_mu_as, l2_mu_ad, l2_mu_b,
           dec_W1, dec_b1, dec_W2, dec_b2):
    r_um = edge_index_um[0].astype(jnp.int32)
    c_um = edge_index_um[1].astype(jnp.int32)
    r_mu = edge_index_mu[0].astype(jnp.int32)
    c_mu = edge_index_mu[1].astype(jnp.int32)
    r_l = edge_label_index[0].astype(jnp.int32)
    c_l = edge_label_index[1].astype(jnp.int32)

    zm = jax.nn.relu(_gat_jnp(x_user, x_movie, r_um, c_um,
                              l1_um_Ws, l1_um_Wd, l1_um_as, l1_um_ad, l1_um_b, H1, C1))
    zu = jax.nn.relu(_gat_jnp(x_movie, x_user, r_mu, c_mu,
                              l1_mu_Ws, l1_mu_Wd, l1_mu_as, l1_mu_ad, l1_mu_b, H1, C1))
    zm2 = _gat_jnp(zu, zm, r_um, c_um,
                   l2_um_Ws, l2_um_Wd, l2_um_as, l2_um_ad, l2_um_b, H2, C2)
    zu2 = _gat_jnp(zm, zu, r_mu, c_mu,
                   l2_mu_Ws, l2_mu_Wd, l2_mu_as, l2_mu_ad, l2_mu_b, H2, C2)
    return _decoder(zu2[r_l], zm2[c_l], dec_W1, dec_b1, dec_W2, dec_b2)


# speedup vs baseline: 14.6844x; 13.9767x over previous
import functools

import jax
import jax.numpy as jnp
from jax import lax
from jax.experimental import pallas as pl
from jax.experimental.pallas import tpu as pltpu
from jax.experimental.pallas import tpu_sc as plsc

H1, C1 = 8, 32
H2, C2 = 4, 256
D1 = H1 * C1
D2 = H2 * C2
HP = 16
WC = 128
NT = 16


def _mm_body(x_ref, w_ref, o_ref):
    o_ref[...] = jnp.dot(x_ref[...], w_ref[...],
                         preferred_element_type=jnp.float32)


def _mm(x, w, tn=512):
    n, k = x.shape
    d = w.shape[1]
    tn = min(tn, n)
    g = -(-n // tn)
    return pl.pallas_call(
        _mm_body,
        grid=(g,),
        in_specs=[pl.BlockSpec((tn, k), lambda i: (i, 0)),
                  pl.BlockSpec((k, d), lambda i: (0, 0))],
        out_specs=pl.BlockSpec((tn, d), lambda i: (i, 0)),
        out_shape=jax.ShapeDtypeStruct((n, d), jnp.float32),
    )(x, w)


def _ex_body(pu_ref, pm_ref, eu_ref, em_ref):
    def f(a):
        return jnp.exp(jnp.where(a >= 0, a, 0.2 * a))
    eu_ref[...] = f(pu_ref[:, :HP] + pm_ref[:, :HP])
    em_ref[...] = f(pu_ref[:, HP:2 * HP] + pm_ref[:, HP:2 * HP])


def _edge_ex(pu, pm, col0, te=8192):
    E = pu.shape[0]
    g = -(-E // te)
    cb = col0 // 128
    return pl.pallas_call(
        _ex_body,
        grid=(g,),
        in_specs=[pl.BlockSpec((te, 128), lambda i: (i, cb)),
                  pl.BlockSpec((te, 128), lambda i: (i, cb))],
        out_specs=[pl.BlockSpec((te, HP), lambda i: (i, 0)),
                   pl.BlockSpec((te, HP), lambda i: (i, 0))],
        out_shape=[jax.ShapeDtypeStruct((E, HP), jnp.float32),
                   jax.ShapeDtypeStruct((E, HP), jnp.float32)],
    )(pu, pm)


def _msg_body(g_ref, ex_ref, r_ref, o_ref):
    o_ref[0] = g_ref[...] * jnp.dot(ex_ref[...], r_ref[...],
                                    preferred_element_type=jnp.float32)


def _msg(G, ex, R, gmod, te=2048):
    E = G.shape[0]
    W = R.shape[1]
    n_chunks = W // WC
    g = -(-E // te)
    return pl.pallas_call(
        _msg_body,
        grid=(g, n_chunks),
        in_specs=[pl.BlockSpec((te, WC), lambda i, j: (i, j % gmod)),
                  pl.BlockSpec((te, HP), lambda i, j: (i, 0)),
                  pl.BlockSpec((HP, WC), lambda i, j: (0, j))],
        out_specs=pl.BlockSpec((1, te, WC), lambda i, j: (j, i, 0)),
        out_shape=jax.ShapeDtypeStruct((n_chunks, E, WC), jnp.float32),
    )(G, ex, R)


def _fin1_body(a_ref, den_ref, r_ref, b_ref, o_ref):
    denr = jnp.dot(den_ref[...], r_ref[...], preferred_element_type=jnp.float32)
    z = a_ref[0] / (denr + 1e-16) + b_ref[...]
    o_ref[...] = jnp.maximum(z, 0.0)


def _finalize1(agg3, den, R, b, tn=1024):
    n_chunks, N, _ = agg3.shape
    g = -(-N // tn)
    return pl.pallas_call(
        _fin1_body,
        grid=(g, n_chunks),
        in_specs=[pl.BlockSpec((1, tn, WC), lambda i, j: (j, i, 0)),
                  pl.BlockSpec((tn, HP), lambda i, j: (i, 0)),
                  pl.BlockSpec((HP, WC), lambda i, j: (0, j)),
                  pl.BlockSpec((1, WC), lambda i, j: (0, j))],
        out_specs=pl.BlockSpec((tn, WC), lambda i, j: (i, j)),
        out_shape=jax.ShapeDtypeStruct((N, n_chunks * WC), jnp.float32),
    )(agg3, den, R, b.reshape(1, -1))


def _fin2_body(a_ref, den_ref, r_ref, w_ref, b_ref, o_ref):
    j = pl.program_id(1)

    @pl.when(j == 0)
    def _():
        o_ref[...] = jnp.broadcast_to(b_ref[...], o_ref.shape)

    denr = jnp.dot(den_ref[...], r_ref[...], preferred_element_type=jnp.float32)
    z = a_ref[0] / (denr + 1e-16)
    o_ref[...] += jnp.dot(z, w_ref[0], preferred_element_type=jnp.float32)


def _finalize2(agg3, den, R, Wp, b, tn=512):
    n_chunks, N, _ = agg3.shape
    g = -(-N // tn)
    return pl.pallas_call(
        _fin2_body,
        grid=(g, n_chunks),
        in_specs=[pl.BlockSpec((1, tn, WC), lambda i, j: (j, i, 0)),
                  pl.BlockSpec((tn, HP), lambda i, j: (i, 0)),
                  pl.BlockSpec((HP, WC), lambda i, j: (0, j)),
                  pl.BlockSpec((1, WC, D2), lambda i, j: (j, 0, 0)),
                  pl.BlockSpec((1, D2), lambda i, j: (0, 0))],
        out_specs=pl.BlockSpec((tn, D2), lambda i, j: (i, 0)),
        out_shape=jax.ShapeDtypeStruct((N, D2), jnp.float32),
    )(agg3, den, R, Wp, b.reshape(1, -1))


def _dec_body(zr_ref, zc_ref, w1a_ref, w1b_ref, b1_ref, w2_ref, b2_ref, o_ref):
    z = (zr_ref[...] @ w1a_ref[...] + zc_ref[...] @ w1b_ref[...]
         + b1_ref[...])
    z = jnp.maximum(z, 0.0)
    o_ref[...] = (z @ w2_ref[...] + b2_ref[...])[:, 0]


def _decoder(zr, zc, W1, b1, W2, b2, tb=2048):
    B = zr.shape[0]
    W1a, W1b = W1[:D2], W1[D2:]
    return pl.pallas_call(
        _dec_body,
        grid=(B // tb,),
        in_specs=[pl.BlockSpec((tb, D2), lambda i: (i, 0)),
                  pl.BlockSpec((tb, D2), lambda i: (i, 0)),
                  pl.BlockSpec((D2, 32), lambda i: (0, 0)),
                  pl.BlockSpec((D2, 32), lambda i: (0, 0)),
                  pl.BlockSpec((32,), lambda i: (0,)),
                  pl.BlockSpec((32, 1), lambda i: (0, 0)),
                  pl.BlockSpec((1,), lambda i: (0,))],
        out_specs=pl.BlockSpec((tb,), lambda i: (i,)),
        out_shape=jax.ShapeDtypeStruct((B,), jnp.float32),
    )(zr, zc, W1a, W1b, b1, W2, b2)


def _sc_gather_pair(ta, ia, tb, ib):
    Da, Db = ta.shape[1], tb.shape[1]
    Ea, Eb = ia.shape[0], ib.shape[0]
    rba = 128 if Da <= 256 else (64 if Da <= 512 else 32)
    rbb = 128 if Db <= 256 else (64 if Db <= 512 else 32)
    mesh = plsc.VectorSubcoreMesh(core_axis_name="c", subcore_axis_name="s")

    def one(t_ref, i_ref, o_ref, ibuf, gbuf, itail, gtail, sem, per, rb, tail):
        s = lax.axis_index("s")
        base = s * per
        nb = per // rb

        def step(k, carry):
            e0 = base + k * rb
            pltpu.sync_copy(i_ref.at[pl.ds(e0, rb)], ibuf)
            pltpu.async_copy(t_ref.at[ibuf], gbuf, sem).wait()
            pltpu.sync_copy(gbuf, o_ref.at[pl.ds(e0, rb)])
            return carry

        lax.fori_loop(0, nb, step, 0)
        if tail:
            e0 = base + nb * rb
            pltpu.sync_copy(i_ref.at[pl.ds(e0, tail)], itail)
            pltpu.async_copy(t_ref.at[itail], gtail, sem).wait()
            pltpu.sync_copy(gtail, o_ref.at[pl.ds(e0, tail)])

    pera, perb = Ea // NT, Eb // NT
    taila, tailb = pera % rba, perb % rbb

    def body(ta_ref, ia_ref, tb_ref, ib_ref, oa_ref, ob_ref,
             ibufa, gbufa, itaila, gtaila,
             ibufb, gbufb, itailb, gtailb, sem):
        core = lax.axis_index("c")

        @pl.when(core == 0)
        def _():
            one(ta_ref, ia_ref, oa_ref, ibufa, gbufa, itaila, gtaila, sem,
                pera, rba, taila)

        @pl.when(core == 1)
        def _():
            one(tb_ref, ib_ref, ob_ref, ibufb, gbufb, itailb, gtailb, sem,
                perb, rbb, tailb)

    f = pl.kernel(
        body,
        out_type=[jax.ShapeDtypeStruct((Ea, Da), jnp.float32),
                  jax.ShapeDtypeStruct((Eb, Db), jnp.float32)],
        mesh=mesh,
        scratch_types=[
            pltpu.VMEM((rba,), jnp.int32), pltpu.VMEM((rba, Da), jnp.float32),
            pltpu.VMEM((max(taila, 8),), jnp.int32),
            pltpu.VMEM((max(taila, 8), Da), jnp.float32),
            pltpu.VMEM((rbb,), jnp.int32), pltpu.VMEM((rbb, Db), jnp.float32),
            pltpu.VMEM((max(tailb, 8),), jnp.int32),
            pltpu.VMEM((max(tailb, 8), Db), jnp.float32),
            pltpu.SemaphoreType.DMA,
        ],
    )
    return f(ta, ia, tb, ib)


def _sc_segsum_pair(Ma, ca, Mb, cb, N):
    n_chunks, E, wc = Ma.shape
    mesh = plsc.VectorSubcoreMesh(core_axis_name="c", subcore_axis_name="s")
    per = E // NT
    nb = per // 128
    tail = per % 128
    rp = -(-N // NT) // 8 * 8
    rl = N - (NT - 1) * rp

    def one(M_ref, c_ref, o_ref, z_ref, mbuf, cbuf, mtail, ctail, acc):
        s = lax.axis_index("s")
        base = s * per

        def chunk_body(j, carry):
            @pl.when(s < NT - 1)
            def _():
                pltpu.sync_copy(z_ref, acc.at[pl.ds(s * rp, rp)])

            @pl.when(s == NT - 1)
            def _():
                pltpu.sync_copy(z_ref.at[pl.ds(0, rl)],
                                acc.at[pl.ds((NT - 1) * rp, rl)])
            plsc.subcore_barrier()

            def step(k, c2):
                e0 = base + k * 128
                pltpu.sync_copy(c_ref.at[pl.ds(e0, 128)], cbuf)
                pltpu.sync_copy(M_ref.at[j, pl.ds(e0, 128)], mbuf)
                pltpu.sync_copy(mbuf, acc.at[cbuf], add=True)
                return c2

            lax.fori_loop(0, nb, step, 0)
            if tail:
                e0 = base + nb * 128
                pltpu.sync_copy(c_ref.at[pl.ds(e0, tail)], ctail)
                pltpu.sync_copy(M_ref.at[j, pl.ds(e0, tail)], mtail)
                pltpu.sync_copy(mtail, acc.at[ctail], add=True)
            plsc.subcore_barrier()

            @pl.when(s < NT - 1)
            def _():
                pltpu.sync_copy(acc.at[pl.ds(s * rp, rp)],
                                o_ref.at[j, pl.ds(s * rp, rp)])

            @pl.when(s == NT - 1)
            def _():
                pltpu.sync_copy(acc.at[pl.ds((NT - 1) * rp, rl)],
                                o_ref.at[j, pl.ds((NT - 1) * rp, rl)])
            return carry

        lax.fori_loop(0, n_chunks, chunk_body, 0)

    def body(Ma_ref, ca_ref, Mb_ref, cb_ref, z_ref, oa_ref, ob_ref,
             mbuf, cbuf, mtail, ctail, acc):
        core = lax.axis_index("c")

        @pl.when(core == 0)
        def _():
            one(Ma_ref, ca_ref, oa_ref, z_ref, mbuf, cbuf, mtail, ctail, acc)

        @pl.when(core == 1)
        def _():
            one(Mb_ref, cb_ref, ob_ref, z_ref, mbuf, cbuf, mtail, ctail, acc)

    zeros = jnp.zeros((rp, wc), jnp.float32)
    f = pl.kernel(
        body,
        out_type=[jax.ShapeDtypeStruct((n_chunks, N, wc), jnp.float32),
                  jax.ShapeDtypeStruct((n_chunks, N, wc), jnp.float32)],
        mesh=mesh,
        scratch_types=[
            pltpu.VMEM((128, wc), jnp.float32),
            pltpu.VMEM((128,), jnp.int32),
            pltpu.VMEM((max(tail, 8), wc), jnp.float32),
            pltpu.VMEM((max(tail, 8),), jnp.int32),
            pltpu.VMEM_SHARED((N, wc), jnp.float32),
        ],
    )
    return f(Ma, ca, Mb, cb, zeros)


def _ablk(a, H, C):
    m = jnp.zeros((H * C, HP), jnp.float32)
    return m.at[jnp.arange(H * C), jnp.arange(H * C) // C].set(a.reshape(-1))


def _rep(H, C, W):
    m = jnp.zeros((HP, W), jnp.float32)
    return m.at[jnp.arange(W) // C, jnp.arange(W)].set(
        jnp.where(jnp.arange(W) // C < H, 1.0, 0.0))


def kernel(x_user, x_movie, edge_index_um, edge_index_mu, edge_label_index,
           l1_um_Ws, l1_um_Wd, l1_um_as, l1_um_ad, l1_um_b,
           l2_um_Ws, l2_um_Wd, l2_um_as, l2_um_ad, l2_um_b,
           l1_mu_Ws, l1_mu_Wd, l1_mu_as, l1_mu_ad, l1_mu_b,
           l2_mu_Ws, l2_mu_Wd, l2_mu_as, l2_mu_ad, l2_mu_b,
           dec_W1, dec_b1, dec_W2, dec_b2):
    r = edge_index_um[0].astype(jnp.int32)
    c = edge_index_um[1].astype(jnp.int32)
    r_l = edge_label_index[0].astype(jnp.int32)
    c_l = edge_label_index[1].astype(jnp.int32)
    N = x_user.shape[0]

    R1 = _rep(H1, C1, D1)
    R2 = _rep(H2, D1, H2 * D1)

    a1s_um = _ablk(l1_um_as, H1, C1)
    a1d_um = _ablk(l1_um_ad, H1, C1)
    a1s_mu = _ablk(l1_mu_as, H1, C1)
    a1d_mu = _ablk(l1_mu_ad, H1, C1)
    pad96 = jnp.zeros((128, 96), jnp.float32)
    wu1 = jnp.concatenate([l1_um_Ws, _mm(l1_um_Ws, a1s_um),
                           _mm(l1_mu_Wd, a1d_mu), pad96], axis=1)
    wm1 = jnp.concatenate([l1_mu_Ws, _mm(l1_um_Wd, a1d_um),
                           _mm(l1_mu_Ws, a1s_mu), pad96], axis=1)
    pu1 = _mm(x_user, wu1)
    pm1 = _mm(x_movie, wm1)

    g1_um, g1_mu = _sc_gather_pair(pu1, r, pm1, c)
    ex1_um, ex1_mu = _edge_ex(g1_um, g1_mu, D1)
    m1_um = _msg(g1_um, ex1_um, R1, 2)
    m1_mu = _msg(g1_mu, ex1_mu, R1, 2)
    den1_um, den1_mu = _sc_segsum_pair(ex1_um.reshape(1, -1, HP), c,
                                       ex1_mu.reshape(1, -1, HP), r, N)
    agg1_um, agg1_mu = _sc_segsum_pair(m1_um, c, m1_mu, r, N)
    zm = _finalize1(agg1_um, den1_um[0], R1, l1_um_b)
    zu = _finalize1(agg1_mu, den1_mu[0], R1, l1_mu_b)

    a2s_um = _ablk(l2_um_as, H2, C2)
    a2d_um = _ablk(l2_um_ad, H2, C2)
    a2s_mu = _ablk(l2_mu_as, H2, C2)
    a2d_mu = _ablk(l2_mu_ad, H2, C2)
    pad96b = jnp.zeros((D1, 96), jnp.float32)
    wu2 = jnp.concatenate([_mm(l2_um_Ws, a2s_um), _mm(l2_mu_Wd, a2d_mu),
                           pad96b], axis=1)
    wm2 = jnp.concatenate([_mm(l2_um_Wd, a2d_um), _mm(l2_mu_Ws, a2s_mu),
                           pad96b], axis=1)
    pu2 = jnp.concatenate([zu, _mm(zu, wu2)], axis=1)
    pm2 = jnp.concatenate([zm, _mm(zm, wm2)], axis=1)

    g2_um, g2_mu = _sc_gather_pair(pu2, r, pm2, c)
    ex2_um, ex2_mu = _edge_ex(g2_um, g2_mu, D1)
    m2_um = _msg(g2_um, ex2_um, R2, 2)
    m2_mu = _msg(g2_mu, ex2_mu, R2, 2)
    den2_um, den2_mu = _sc_segsum_pair(ex2_um.reshape(1, -1, HP), c,
                                       ex2_mu.reshape(1, -1, HP), r, N)
    agg2_um, agg2_mu = _sc_segsum_pair(m2_um, c, m2_mu, r, N)

    def _wp(Ws):
        blocks = []
        for j in range(H2 * D1 // WC):
            h, part = j // 2, j % 2
            blk = jnp.zeros((WC, D2), jnp.float32)
            blk = blk.at[:, h * C2:(h + 1) * C2].set(
                Ws[part * WC:(part + 1) * WC, h * C2:(h + 1) * C2])
            blocks.append(blk)
        return jnp.stack(blocks)

    zm2 = _finalize2(agg2_um, den2_um[0], R2, _wp(l2_um_Ws), l2_um_b)
    zu2 = _finalize2(agg2_mu, den2_mu[0], R2, _wp(l2_mu_Ws), l2_mu_b)

    dr, dc = _sc_gather_pair(zu2, r_l, zm2, c_l)
    return _decoder(dr, dc, dec_W1, dec_b1, dec_W2, dec_b2)


# speedup vs baseline: 15.3986x; 1.0486x over previous
import functools

import jax
import jax.numpy as jnp
from jax import lax
from jax.experimental import pallas as pl
from jax.experimental.pallas import tpu as pltpu
from jax.experimental.pallas import tpu_sc as plsc

H1, C1 = 8, 32
H2, C2 = 4, 256
D1 = H1 * C1
D2 = H2 * C2
HP = 16
HD = 8
WC = 128
NT = 16


def _mm_body(x_ref, w_ref, o_ref):
    o_ref[...] = jnp.dot(x_ref[...], w_ref[...],
                         preferred_element_type=jnp.float32)


def _mm(x, w, tn=512):
    n, k = x.shape
    d = w.shape[1]
    tn = min(tn, n)
    g = -(-n // tn)
    return pl.pallas_call(
        _mm_body,
        grid=(g,),
        in_specs=[pl.BlockSpec((tn, k), lambda i: (i, 0)),
                  pl.BlockSpec((k, d), lambda i: (0, 0))],
        out_specs=pl.BlockSpec((tn, d), lambda i: (i, 0)),
        out_shape=jax.ShapeDtypeStruct((n, d), jnp.float32),
    )(x, w)


def _ex_body(pu_ref, pm_ref, eu_ref, em_ref):
    def f(a):
        return jnp.exp(jnp.where(a >= 0, a, 0.2 * a))
    eu_ref[...] = f(pu_ref[:, :HP] + pm_ref[:, :HP])
    em_ref[...] = f(pu_ref[:, HP:2 * HP] + pm_ref[:, HP:2 * HP])


def _edge_ex(pu, pm, col0, te=8192):
    E = pu.shape[0]
    g = -(-E // te)
    cb = col0 // 128
    return pl.pallas_call(
        _ex_body,
        grid=(g,),
        in_specs=[pl.BlockSpec((te, 128), lambda i: (i, cb)),
                  pl.BlockSpec((te, 128), lambda i: (i, cb))],
        out_specs=[pl.BlockSpec((te, HP), lambda i: (i, 0)),
                   pl.BlockSpec((te, HP), lambda i: (i, 0))],
        out_shape=[jax.ShapeDtypeStruct((E, HP), jnp.float32),
                   jax.ShapeDtypeStruct((E, HP), jnp.float32)],
    )(pu, pm)


def _msg_body(g_ref, ex_ref, r_ref, o_ref):
    o_ref[0] = g_ref[...] * jnp.dot(ex_ref[...], r_ref[...],
                                    preferred_element_type=jnp.float32)


def _msg(G, ex, R, gmod, te=2048):
    E = G.shape[0]
    W = R.shape[1]
    n_chunks = W // WC
    g = -(-E // te)
    return pl.pallas_call(
        _msg_body,
        grid=(g, n_chunks),
        in_specs=[pl.BlockSpec((te, WC), lambda i, j: (i, j % gmod)),
                  pl.BlockSpec((te, HP), lambda i, j: (i, 0)),
                  pl.BlockSpec((HP, WC), lambda i, j: (0, j))],
        out_specs=pl.BlockSpec((1, te, WC), lambda i, j: (j, i, 0)),
        out_shape=jax.ShapeDtypeStruct((n_chunks, E, WC), jnp.float32),
    )(G, ex, R)


def _fin1_body(a_ref, den_ref, r_ref, b_ref, o_ref):
    denr = jnp.dot(den_ref[...], r_ref[...], preferred_element_type=jnp.float32)
    z = a_ref[0] / (denr + 1e-16) + b_ref[...]
    o_ref[...] = jnp.maximum(z, 0.0)


def _finalize1(agg3, den, R, b, tn=1024):
    n_chunks, N, _ = agg3.shape
    g = -(-N // tn)
    return pl.pallas_call(
        _fin1_body,
        grid=(g, n_chunks),
        in_specs=[pl.BlockSpec((1, tn, WC), lambda i, j: (j, i, 0)),
                  pl.BlockSpec((tn, HP), lambda i, j: (i, 0)),
                  pl.BlockSpec((HP, WC), lambda i, j: (0, j)),
                  pl.BlockSpec((1, WC), lambda i, j: (0, j))],
        out_specs=pl.BlockSpec((tn, WC), lambda i, j: (i, j)),
        out_shape=jax.ShapeDtypeStruct((N, n_chunks * WC), jnp.float32),
    )(agg3, den, R, b.reshape(1, -1))


def _fin2_body(a_ref, den_ref, r_ref, w_ref, b_ref, o_ref):
    j = pl.program_id(1)

    @pl.when(j == 0)
    def _():
        o_ref[...] = jnp.broadcast_to(b_ref[...], o_ref.shape)

    denr = jnp.dot(den_ref[...], r_ref[...], preferred_element_type=jnp.float32)
    z = a_ref[0] / (denr + 1e-16)
    o_ref[...] += jnp.dot(z, w_ref[0], preferred_element_type=jnp.float32)


def _finalize2(agg3, den, R, Wp, b, tn=512):
    n_chunks, N, _ = agg3.shape
    g = -(-N // tn)
    return pl.pallas_call(
        _fin2_body,
        grid=(g, n_chunks),
        in_specs=[pl.BlockSpec((1, tn, WC), lambda i, j: (j, i, 0)),
                  pl.BlockSpec((tn, HP), lambda i, j: (i, 0)),
                  pl.BlockSpec((HP, WC), lambda i, j: (0, j)),
                  pl.BlockSpec((1, WC, D2), lambda i, j: (j, 0, 0)),
                  pl.BlockSpec((1, D2), lambda i, j: (0, 0))],
        out_specs=pl.BlockSpec((tn, D2), lambda i, j: (i, 0)),
        out_shape=jax.ShapeDtypeStruct((N, D2), jnp.float32),
    )(agg3, den, R, Wp, b.reshape(1, -1))


def _dec_body(pr_ref, pc_ref, b1_ref, w2_ref, b2_ref, o_ref):
    z = pr_ref[:, :32] + pc_ref[:, :32] + b1_ref[...]
    z = jnp.maximum(z, 0.0)
    o_ref[...] = (z @ w2_ref[...] + b2_ref[...])[:, 0]


def _decoder(prg, pcg, b1, W2, b2, tb=4096):
    B = prg.shape[0]
    return pl.pallas_call(
        _dec_body,
        grid=(B // tb,),
        in_specs=[pl.BlockSpec((tb, 128), lambda i: (i, 0)),
                  pl.BlockSpec((tb, 128), lambda i: (i, 0)),
                  pl.BlockSpec((32,), lambda i: (0,)),
                  pl.BlockSpec((32, 1), lambda i: (0, 0)),
                  pl.BlockSpec((1,), lambda i: (0,))],
        out_specs=pl.BlockSpec((tb,), lambda i: (i,)),
        out_shape=jax.ShapeDtypeStruct((B,), jnp.float32),
    )(prg, pcg, b1, W2, b2)


def _sc_gather_pair(ta, ia, tb, ib):
    Da, Db = ta.shape[1], tb.shape[1]
    Ea, Eb = ia.shape[0], ib.shape[0]
    rba = 128 if Da <= 256 else (64 if Da <= 512 else 32)
    rbb = 128 if Db <= 256 else (64 if Db <= 512 else 32)
    mesh = plsc.VectorSubcoreMesh(core_axis_name="c", subcore_axis_name="s")

    def one(t_ref, i_ref, o_ref, ibuf, gbuf, itail, gtail, sem, per, rb, tail):
        s = lax.axis_index("s")
        base = s * per
        nb = per // rb

        def step(k, carry):
            e0 = base + k * rb
            pltpu.sync_copy(i_ref.at[pl.ds(e0, rb)], ibuf)
            pltpu.async_copy(t_ref.at[ibuf], gbuf, sem).wait()
            pltpu.sync_copy(gbuf, o_ref.at[pl.ds(e0, rb)])
            return carry

        lax.fori_loop(0, nb, step, 0)
        if tail:
            e0 = base + nb * rb
            pltpu.sync_copy(i_ref.at[pl.ds(e0, tail)], itail)
            pltpu.async_copy(t_ref.at[itail], gtail, sem).wait()
            pltpu.sync_copy(gtail, o_ref.at[pl.ds(e0, tail)])

    pera, perb = Ea // NT, Eb // NT
    taila, tailb = pera % rba, perb % rbb

    def body(ta_ref, ia_ref, tb_ref, ib_ref, oa_ref, ob_ref,
             ibufa, gbufa, itaila, gtaila,
             ibufb, gbufb, itailb, gtailb, sem):
        core = lax.axis_index("c")

        @pl.when(core == 0)
        def _():
            one(ta_ref, ia_ref, oa_ref, ibufa, gbufa, itaila, gtaila, sem,
                pera, rba, taila)

        @pl.when(core == 1)
        def _():
            one(tb_ref, ib_ref, ob_ref, ibufb, gbufb, itailb, gtailb, sem,
                perb, rbb, tailb)

    f = pl.kernel(
        body,
        out_type=[jax.ShapeDtypeStruct((Ea, Da), jnp.float32),
                  jax.ShapeDtypeStruct((Eb, Db), jnp.float32)],
        mesh=mesh,
        scratch_types=[
            pltpu.VMEM((rba,), jnp.int32), pltpu.VMEM((rba, Da), jnp.float32),
            pltpu.VMEM((max(taila, 8),), jnp.int32),
            pltpu.VMEM((max(taila, 8), Da), jnp.float32),
            pltpu.VMEM((rbb,), jnp.int32), pltpu.VMEM((rbb, Db), jnp.float32),
            pltpu.VMEM((max(tailb, 8),), jnp.int32),
            pltpu.VMEM((max(tailb, 8), Db), jnp.float32),
            pltpu.SemaphoreType.DMA,
        ],
    )
    return f(ta, ia, tb, ib)


def _sc_segsum_pair(Ma, ca, Mb, cb, N):
    n_chunks, E, wc = Ma.shape
    mesh = plsc.VectorSubcoreMesh(core_axis_name="c", subcore_axis_name="s")
    per = E // NT
    nb = per // 128
    tail = per % 128
    rp = -(-N // NT) // 8 * 8
    rl = N - (NT - 1) * rp

    def one(M_ref, c_ref, o_ref, z_ref, mbuf, cbuf, mtail, ctail, acc):
        s = lax.axis_index("s")
        base = s * per

        def rows_split(fn):
            @pl.when(s < NT - 1)
            def _():
                fn(s * rp, rp)

            @pl.when(s == NT - 1)
            def _():
                fn((NT - 1) * rp, rl)

        def chunk_body(j, carry):
            rows_split(lambda r0, nr: pltpu.sync_copy(
                z_ref.at[pl.ds(0, nr)], acc.at[pl.ds(r0, nr)]))
            plsc.subcore_barrier()

            def step(k, c2):
                e0 = base + k * 128
                pltpu.sync_copy(c_ref.at[pl.ds(e0, 128)], cbuf)
                pltpu.sync_copy(M_ref.at[j, pl.ds(e0, 128)], mbuf)
                pltpu.sync_copy(mbuf, acc.at[cbuf], add=True)
                return c2

            lax.fori_loop(0, nb, step, 0)
            if tail:
                e0 = base + nb * 128
                pltpu.sync_copy(c_ref.at[pl.ds(e0, tail)], ctail)
                pltpu.sync_copy(M_ref.at[j, pl.ds(e0, tail)], mtail)
                pltpu.sync_copy(mtail, acc.at[ctail], add=True)
            plsc.subcore_barrier()
            rows_split(lambda r0, nr: pltpu.sync_copy(
                acc.at[pl.ds(r0, nr)], o_ref.at[j, pl.ds(r0, nr)]))
            return carry

        lax.fori_loop(0, n_chunks, chunk_body, 0)

    def body(Ma_ref, ca_ref, Mb_ref, cb_ref, z_ref, oa_ref, ob_ref,
             mbuf, cbuf, mtail, ctail, acc):
        core = lax.axis_index("c")

        @pl.when(core == 0)
        def _():
            one(Ma_ref, ca_ref, oa_ref, z_ref, mbuf, cbuf, mtail, ctail, acc)

        @pl.when(core == 1)
        def _():
            one(Mb_ref, cb_ref, ob_ref, z_ref, mbuf, cbuf, mtail, ctail, acc)

    zeros = jnp.zeros((rp, wc), jnp.float32)
    f = pl.kernel(
        body,
        out_type=[jax.ShapeDtypeStruct((n_chunks, N, wc), jnp.float32),
                  jax.ShapeDtypeStruct((n_chunks, N, wc), jnp.float32)],
        mesh=mesh,
        scratch_types=[
            pltpu.VMEM((128, wc), jnp.float32),
            pltpu.VMEM((128,), jnp.int32),
            pltpu.VMEM((max(tail, 8), wc), jnp.float32),
            pltpu.VMEM((max(tail, 8),), jnp.int32),
            pltpu.VMEM_SHARED((N, wc), jnp.float32),
        ],
    )
    return f(Ma, ca, Mb, cb, zeros)


def _ablk(a, H, C):
    m = jnp.zeros((H * C, HP), jnp.float32)
    return m.at[jnp.arange(H * C), jnp.arange(H * C) // C].set(a.reshape(-1))


def _rep(H, C, W):
    m = jnp.zeros((HP, W), jnp.float32)
    return m.at[jnp.arange(W) // C, jnp.arange(W)].set(
        jnp.where(jnp.arange(W) // C < H, 1.0, 0.0))


def kernel(x_user, x_movie, edge_index_um, edge_index_mu, edge_label_index,
           l1_um_Ws, l1_um_Wd, l1_um_as, l1_um_ad, l1_um_b,
           l2_um_Ws, l2_um_Wd, l2_um_as, l2_um_ad, l2_um_b,
           l1_mu_Ws, l1_mu_Wd, l1_mu_as, l1_mu_ad, l1_mu_b,
           l2_mu_Ws, l2_mu_Wd, l2_mu_as, l2_mu_ad, l2_mu_b,
           dec_W1, dec_b1, dec_W2, dec_b2):
    r = edge_index_um[0].astype(jnp.int32)
    c = edge_index_um[1].astype(jnp.int32)
    r_l = edge_label_index[0].astype(jnp.int32)
    c_l = edge_label_index[1].astype(jnp.int32)
    N = x_user.shape[0]

    R1 = _rep(H1, C1, D1)
    R2 = _rep(H2, D1, H2 * D1)

    a1s_um = _ablk(l1_um_as, H1, C1)
    a1d_um = _ablk(l1_um_ad, H1, C1)
    a1s_mu = _ablk(l1_mu_as, H1, C1)
    a1d_mu = _ablk(l1_mu_ad, H1, C1)
    pad96 = jnp.zeros((128, 96), jnp.float32)
    wu1 = jnp.concatenate([l1_um_Ws, _mm(l1_um_Ws, a1s_um),
                           _mm(l1_mu_Wd, a1d_mu), pad96], axis=1)
    wm1 = jnp.concatenate([l1_mu_Ws, _mm(l1_um_Wd, a1d_um),
                           _mm(l1_mu_Ws, a1s_mu), pad96], axis=1)
    pu1 = _mm(x_user, wu1)
    pm1 = _mm(x_movie, wm1)

    g1_um, g1_mu = _sc_gather_pair(pu1, r, pm1, c)
    ex1_um, ex1_mu = _edge_ex(g1_um, g1_mu, D1)
    m1_um = _msg(g1_um, ex1_um, R1, 2)
    m1_mu = _msg(g1_mu, ex1_mu, R1, 2)
    den1_um, den1_mu = _sc_segsum_pair(ex1_um.reshape(1, -1, HP), c,
                                       ex1_mu.reshape(1, -1, HP), r, N)
    agg1_um, agg1_mu = _sc_segsum_pair(m1_um, c, m1_mu, r, N)
    zm = _finalize1(agg1_um, den1_um[0], R1, l1_um_b)
    zu = _finalize1(agg1_mu, den1_mu[0], R1, l1_mu_b)

    a2s_um = _ablk(l2_um_as, H2, C2)
    a2d_um = _ablk(l2_um_ad, H2, C2)
    a2s_mu = _ablk(l2_mu_as, H2, C2)
    a2d_mu = _ablk(l2_mu_ad, H2, C2)
    pad96b = jnp.zeros((D1, 96), jnp.float32)
    wu2 = jnp.concatenate([_mm(l2_um_Ws, a2s_um), _mm(l2_mu_Wd, a2d_mu),
                           pad96b], axis=1)
    wm2 = jnp.concatenate([_mm(l2_um_Wd, a2d_um), _mm(l2_mu_Ws, a2s_mu),
                           pad96b], axis=1)
    pu2 = jnp.concatenate([zu, _mm(zu, wu2)], axis=1)
    pm2 = jnp.concatenate([zm, _mm(zm, wm2)], axis=1)

    g2_um, g2_mu = _sc_gather_pair(pu2, r, pm2, c)
    ex2_um, ex2_mu = _edge_ex(g2_um, g2_mu, D1)
    m2_um = _msg(g2_um, ex2_um, R2, 2)
    m2_mu = _msg(g2_mu, ex2_mu, R2, 2)
    den2_um, den2_mu = _sc_segsum_pair(ex2_um.reshape(1, -1, HP), c,
                                       ex2_mu.reshape(1, -1, HP), r, N)
    agg2_um, agg2_mu = _sc_segsum_pair(m2_um, c, m2_mu, r, N)

    def _wp(Ws):
        blocks = []
        for j in range(H2 * D1 // WC):
            h, part = j // 2, j % 2
            blk = jnp.zeros((WC, D2), jnp.float32)
            blk = blk.at[:, h * C2:(h + 1) * C2].set(
                Ws[part * WC:(part + 1) * WC, h * C2:(h + 1) * C2])
            blocks.append(blk)
        return jnp.stack(blocks)

    zm2 = _finalize2(agg2_um, den2_um[0], R2, _wp(l2_um_Ws), l2_um_b)
    zu2 = _finalize2(agg2_mu, den2_mu[0], R2, _wp(l2_mu_Ws), l2_mu_b)

    w1pad = jnp.zeros((D2, 96), jnp.float32)
    pr = _mm(zu2, jnp.concatenate([dec_W1[:D2], w1pad], axis=1))
    pc = _mm(zm2, jnp.concatenate([dec_W1[D2:], w1pad], axis=1))
    dr, dc = _sc_gather_pair(pr, r_l, pc, c_l)
    return _decoder(dr, dc, dec_b1, dec_W2, dec_b2)


# speedup vs baseline: 20.3472x; 1.3214x over previous
import functools

import jax
import jax.numpy as jnp
from jax import lax
from jax.experimental import pallas as pl
from jax.experimental.pallas import tpu as pltpu
from jax.experimental.pallas import tpu_sc as plsc

H1, C1 = 8, 32
H2, C2 = 4, 256
D1 = H1 * C1
D2 = H2 * C2
HP = 16
HD = 8
WC = 128
NT = 16


def _mm_body(x_ref, w_ref, o_ref):
    o_ref[...] = jnp.dot(x_ref[...], w_ref[...],
                         preferred_element_type=jnp.float32)


def _mm(x, w, tn=512):
    n, k = x.shape
    d = w.shape[1]
    tn = min(tn, n)
    g = -(-n // tn)
    return pl.pallas_call(
        _mm_body,
        grid=(g,),
        in_specs=[pl.BlockSpec((tn, k), lambda i: (i, 0)),
                  pl.BlockSpec((k, d), lambda i: (0, 0))],
        out_specs=pl.BlockSpec((tn, d), lambda i: (i, 0)),
        out_shape=jax.ShapeDtypeStruct((n, d), jnp.float32),
    )(x, w)


def _ex_body(pu_ref, pm_ref, eu_ref, em_ref):
    def f(a):
        return jnp.exp(jnp.where(a >= 0, a, 0.2 * a))
    eu_ref[...] = f(pu_ref[:, :HP] + pm_ref[:, :HP])
    em_ref[...] = f(pu_ref[:, HP:2 * HP] + pm_ref[:, HP:2 * HP])


def _edge_ex(pu, pm, col0, te=8192):
    E = pu.shape[0]
    g = -(-E // te)
    cb = col0 // 128
    return pl.pallas_call(
        _ex_body,
        grid=(g,),
        in_specs=[pl.BlockSpec((te, 128), lambda i: (i, cb)),
                  pl.BlockSpec((te, 128), lambda i: (i, cb))],
        out_specs=[pl.BlockSpec((te, HP), lambda i: (i, 0)),
                   pl.BlockSpec((te, HP), lambda i: (i, 0))],
        out_shape=[jax.ShapeDtypeStruct((E, HP), jnp.float32),
                   jax.ShapeDtypeStruct((E, HP), jnp.float32)],
    )(pu, pm)


def _msg_body(g_ref, ex_ref, r_ref, o_ref):
    o_ref[0] = g_ref[...] * jnp.dot(ex_ref[...], r_ref[...],
                                    preferred_element_type=jnp.float32)


def _msg(G, ex, R, gmod, te=2048):
    E = G.shape[0]
    W = R.shape[1]
    n_chunks = W // WC
    g = -(-E // te)
    return pl.pallas_call(
        _msg_body,
        grid=(g, n_chunks),
        in_specs=[pl.BlockSpec((te, WC), lambda i, j: (i, j % gmod)),
                  pl.BlockSpec((te, HP), lambda i, j: (i, 0)),
                  pl.BlockSpec((HP, WC), lambda i, j: (0, j))],
        out_specs=pl.BlockSpec((1, te, WC), lambda i, j: (j, i, 0)),
        out_shape=jax.ShapeDtypeStruct((n_chunks, E, WC), jnp.float32),
    )(G, ex, R)


def _fin1_body(a_ref, den_ref, r_ref, b_ref, o_ref):
    denr = jnp.dot(den_ref[...], r_ref[...], preferred_element_type=jnp.float32)
    z = a_ref[0] / (denr + 1e-16) + b_ref[...]
    o_ref[...] = jnp.maximum(z, 0.0)


def _finalize1(agg3, den, R, b, tn=1024):
    n_chunks, N, _ = agg3.shape
    g = -(-N // tn)
    return pl.pallas_call(
        _fin1_body,
        grid=(g, n_chunks),
        in_specs=[pl.BlockSpec((1, tn, WC), lambda i, j: (j, i, 0)),
                  pl.BlockSpec((tn, HP), lambda i, j: (i, 0)),
                  pl.BlockSpec((HP, WC), lambda i, j: (0, j)),
                  pl.BlockSpec((1, WC), lambda i, j: (0, j))],
        out_specs=pl.BlockSpec((tn, WC), lambda i, j: (i, j)),
        out_shape=jax.ShapeDtypeStruct((N, n_chunks * WC), jnp.float32),
    )(agg3, den, R, b.reshape(1, -1))


def _fin2_body(a_ref, den_ref, r_ref, w_ref, b_ref, o_ref):
    j = pl.program_id(1)

    @pl.when(j == 0)
    def _():
        o_ref[...] = jnp.broadcast_to(b_ref[...], o_ref.shape)

    denr = jnp.dot(den_ref[...], r_ref[...], preferred_element_type=jnp.float32)
    z = a_ref[0] / (denr + 1e-16)
    o_ref[...] += jnp.dot(z, w_ref[0], preferred_element_type=jnp.float32)


def _finalize2(agg3, den, R, Wp, b, tn=512):
    n_chunks, N, _ = agg3.shape
    g = -(-N // tn)
    return pl.pallas_call(
        _fin2_body,
        grid=(g, n_chunks),
        in_specs=[pl.BlockSpec((1, tn, WC), lambda i, j: (j, i, 0)),
                  pl.BlockSpec((tn, HP), lambda i, j: (i, 0)),
                  pl.BlockSpec((HP, WC), lambda i, j: (0, j)),
                  pl.BlockSpec((1, WC, D2), lambda i, j: (j, 0, 0)),
                  pl.BlockSpec((1, D2), lambda i, j: (0, 0))],
        out_specs=pl.BlockSpec((tn, D2), lambda i, j: (i, 0)),
        out_shape=jax.ShapeDtypeStruct((N, D2), jnp.float32),
    )(agg3, den, R, Wp, b.reshape(1, -1))


def _dec_body(pr_ref, pc_ref, b1_ref, w2_ref, b2_ref, o_ref):
    z = pr_ref[:, :32] + pc_ref[:, :32] + b1_ref[...]
    z = jnp.maximum(z, 0.0)
    o_ref[...] = (z @ w2_ref[...] + b2_ref[...])[:, 0]


def _decoder(prg, pcg, b1, W2, b2, tb=4096):
    B = prg.shape[0]
    return pl.pallas_call(
        _dec_body,
        grid=(B // tb,),
        in_specs=[pl.BlockSpec((tb, 128), lambda i: (i, 0)),
                  pl.BlockSpec((tb, 128), lambda i: (i, 0)),
                  pl.BlockSpec((32,), lambda i: (0,)),
                  pl.BlockSpec((32, 1), lambda i: (0, 0)),
                  pl.BlockSpec((1,), lambda i: (0,))],
        out_specs=pl.BlockSpec((tb,), lambda i: (i,)),
        out_shape=jax.ShapeDtypeStruct((B,), jnp.float32),
    )(prg, pcg, b1, W2, b2)


def _sc_gather_pair(ta, ia, tb, ib):
    Da, Db = ta.shape[1], tb.shape[1]
    Ea, Eb = ia.shape[0], ib.shape[0]
    rba = 128 if Da <= 256 else (64 if Da <= 512 else 32)
    rbb = 128 if Db <= 256 else (64 if Db <= 512 else 32)
    mesh = plsc.VectorSubcoreMesh(core_axis_name="c", subcore_axis_name="s")

    def one(t_ref, i_ref, o_ref, ibufs, gbufs, itail, gtail,
            gsems, wsems, per, rb, tail):
        s = lax.axis_index("s")
        base = s * per
        nb = per // rb

        def idx_load(k, ibuf):
            pltpu.sync_copy(i_ref.at[pl.ds(base + k * rb, rb)], ibuf)

        def gather(k, p):
            pltpu.async_copy(t_ref.at[ibufs[p]], gbufs[p], gsems[p])

        def wb_start(k, p):
            pltpu.async_copy(gbufs[p], o_ref.at[pl.ds(base + k * rb, rb)],
                             wsems[p])

        def wait_g(p):
            pltpu.make_async_copy(t_ref.at[ibufs[p]], gbufs[p],
                                  gsems[p]).wait()

        def wait_w(k, p):
            pltpu.make_async_copy(gbufs[p], o_ref.at[pl.ds(base + k * rb, rb)],
                                  wsems[p]).wait()

        idx_load(0, ibufs[0])
        gather(0, 0)

        def phase(k, p):
            @pl.when(k + 1 < nb)
            def _():
                @pl.when(k >= 1)
                def _():
                    wait_w(k - 1, 1 - p)
                idx_load(k + 1, ibufs[1 - p])
                gather(k + 1, 1 - p)
            wait_g(p)
            wb_start(k, p)

        def step2(i, carry):
            phase(2 * i, 0)
            phase(2 * i + 1, 1)
            return carry

        lax.fori_loop(0, nb // 2, step2, 0)
        wait_w(nb - 2, 0)
        wait_w(nb - 1, 1)
        if tail:
            e0 = base + nb * rb
            pltpu.sync_copy(i_ref.at[pl.ds(e0, tail)], itail)
            pltpu.async_copy(t_ref.at[itail], gtail, gsems[0]).wait()
            pltpu.sync_copy(gtail, o_ref.at[pl.ds(e0, tail)])

    pera, perb = Ea // NT, Eb // NT
    taila, tailb = pera % rba, perb % rbb
    assert (pera // rba) % 2 == 0 and (perb // rbb) % 2 == 0

    def body(ta_ref, ia_ref, tb_ref, ib_ref, oa_ref, ob_ref,
             ia0, ia1, ga0, ga1, ita, gta,
             ib0, ib1, gb0, gb1, itb, gtb,
             sg0, sg1, sw0, sw1):
        core = lax.axis_index("c")

        @pl.when(core == 0)
        def _():
            one(ta_ref, ia_ref, oa_ref, (ia0, ia1), (ga0, ga1), ita, gta,
                (sg0, sg1), (sw0, sw1), pera, rba, taila)

        @pl.when(core == 1)
        def _():
            one(tb_ref, ib_ref, ob_ref, (ib0, ib1), (gb0, gb1), itb, gtb,
                (sg0, sg1), (sw0, sw1), perb, rbb, tailb)

    f = pl.kernel(
        body,
        out_type=[jax.ShapeDtypeStruct((Ea, Da), jnp.float32),
                  jax.ShapeDtypeStruct((Eb, Db), jnp.float32)],
        mesh=mesh,
        scratch_types=[
            pltpu.VMEM((rba,), jnp.int32), pltpu.VMEM((rba,), jnp.int32),
            pltpu.VMEM((rba, Da), jnp.float32),
            pltpu.VMEM((rba, Da), jnp.float32),
            pltpu.VMEM((max(taila, 8),), jnp.int32),
            pltpu.VMEM((max(taila, 8), Da), jnp.float32),
            pltpu.VMEM((rbb,), jnp.int32), pltpu.VMEM((rbb,), jnp.int32),
            pltpu.VMEM((rbb, Db), jnp.float32),
            pltpu.VMEM((rbb, Db), jnp.float32),
            pltpu.VMEM((max(tailb, 8),), jnp.int32),
            pltpu.VMEM((max(tailb, 8), Db), jnp.float32),
            pltpu.SemaphoreType.DMA, pltpu.SemaphoreType.DMA,
            pltpu.SemaphoreType.DMA, pltpu.SemaphoreType.DMA,
        ],
    )
    return f(ta, ia, tb, ib)


def _sc_segsum_pair(Ma, ca, Mb, cb, N):
    n_chunks, E, wc = Ma.shape
    mesh = plsc.VectorSubcoreMesh(core_axis_name="c", subcore_axis_name="s")
    per = E // NT
    nb = per // 128
    tail = per % 128
    assert nb % 2 == 0
    rp = -(-N // NT) // 8 * 8
    rl = N - (NT - 1) * rp

    def one(M_ref, c_ref, o_ref, z_ref, mbufs, cbufs, mtail, ctail, acc,
            lsems, ssems):
        s = lax.axis_index("s")
        base = s * per

        def rows_split(fn):
            @pl.when(s < NT - 1)
            def _():
                fn(s * rp, rp)

            @pl.when(s == NT - 1)
            def _():
                fn((NT - 1) * rp, rl)

        def chunk_body(j, carry):
            rows_split(lambda r0, nr: pltpu.sync_copy(
                z_ref.at[pl.ds(0, nr)], acc.at[pl.ds(r0, nr)]))
            plsc.subcore_barrier()

            def loads(k, p):
                e0 = base + k * 128
                pltpu.async_copy(c_ref.at[pl.ds(e0, 128)], cbufs[p], lsems[p])
                pltpu.async_copy(M_ref.at[j, pl.ds(e0, 128)], mbufs[p],
                                 lsems[p])

            def wait_loads(k, p):
                e0 = base + k * 128
                pltpu.make_async_copy(c_ref.at[pl.ds(e0, 128)], cbufs[p],
                                      lsems[p]).wait()
                pltpu.make_async_copy(M_ref.at[j, pl.ds(e0, 128)], mbufs[p],
                                      lsems[p]).wait()

            def scat(p):
                pltpu.async_copy(mbufs[p], acc.at[cbufs[p]], ssems[p],
                                 add=True)

            def wait_scat(p):
                pltpu.make_async_copy(mbufs[p], acc.at[cbufs[p]],
                                      ssems[p]).wait()

            loads(0, 0)

            def phase(k, p):
                @pl.when(k + 1 < nb)
                def _():
                    @pl.when(k >= 1)
                    def _():
                        wait_scat(1 - p)
                    loads(k + 1, 1 - p)
                wait_loads(k, p)
                scat(p)

            def step2(i, c2):
                phase(2 * i, 0)
                phase(2 * i + 1, 1)
                return c2

            lax.fori_loop(0, nb // 2, step2, 0)
            wait_scat(0)
            wait_scat(1)
            if tail:
                e0 = base + nb * 128
                pltpu.sync_copy(c_ref.at[pl.ds(e0, tail)], ctail)
                pltpu.sync_copy(M_ref.at[j, pl.ds(e0, tail)], mtail)
                pltpu.sync_copy(mtail, acc.at[ctail], add=True)
            plsc.subcore_barrier()
            rows_split(lambda r0, nr: pltpu.sync_copy(
                acc.at[pl.ds(r0, nr)], o_ref.at[j, pl.ds(r0, nr)]))
            return carry

        lax.fori_loop(0, n_chunks, chunk_body, 0)

    def body(Ma_ref, ca_ref, Mb_ref, cb_ref, z_ref, oa_ref, ob_ref,
             mb0, mb1, cb0, cb1, mtail, ctail, acc, sl0, sl1, ss0, ss1):
        core = lax.axis_index("c")

        @pl.when(core == 0)
        def _():
            one(Ma_ref, ca_ref, oa_ref, z_ref, (mb0, mb1), (cb0, cb1),
                mtail, ctail, acc, (sl0, sl1), (ss0, ss1))

        @pl.when(core == 1)
        def _():
            one(Mb_ref, cb_ref, ob_ref, z_ref, (mb0, mb1), (cb0, cb1),
                mtail, ctail, acc, (sl0, sl1), (ss0, ss1))

    zeros = jnp.zeros((rp, wc), jnp.float32)
    f = pl.kernel(
        body,
        out_type=[jax.ShapeDtypeStruct((n_chunks, N, wc), jnp.float32),
                  jax.ShapeDtypeStruct((n_chunks, N, wc), jnp.float32)],
        mesh=mesh,
        scratch_types=[
            pltpu.VMEM((128, wc), jnp.float32),
            pltpu.VMEM((128, wc), jnp.float32),
            pltpu.VMEM((128,), jnp.int32),
            pltpu.VMEM((128,), jnp.int32),
            pltpu.VMEM((max(tail, 8), wc), jnp.float32),
            pltpu.VMEM((max(tail, 8),), jnp.int32),
            pltpu.VMEM_SHARED((N, wc), jnp.float32),
            pltpu.SemaphoreType.DMA, pltpu.SemaphoreType.DMA,
            pltpu.SemaphoreType.DMA, pltpu.SemaphoreType.DMA,
        ],
    )
    return f(Ma, ca, Mb, cb, zeros)


def _ablk(a, H, C):
    m = jnp.zeros((H * C, HP), jnp.float32)
    return m.at[jnp.arange(H * C), jnp.arange(H * C) // C].set(a.reshape(-1))


def _rep(H, C, W):
    m = jnp.zeros((HP, W), jnp.float32)
    return m.at[jnp.arange(W) // C, jnp.arange(W)].set(
        jnp.where(jnp.arange(W) // C < H, 1.0, 0.0))


def kernel(x_user, x_movie, edge_index_um, edge_index_mu, edge_label_index,
           l1_um_Ws, l1_um_Wd, l1_um_as, l1_um_ad, l1_um_b,
           l2_um_Ws, l2_um_Wd, l2_um_as, l2_um_ad, l2_um_b,
           l1_mu_Ws, l1_mu_Wd, l1_mu_as, l1_mu_ad, l1_mu_b,
           l2_mu_Ws, l2_mu_Wd, l2_mu_as, l2_mu_ad, l2_mu_b,
           dec_W1, dec_b1, dec_W2, dec_b2):
    r = edge_index_um[0].astype(jnp.int32)
    c = edge_index_um[1].astype(jnp.int32)
    r_l = edge_label_index[0].astype(jnp.int32)
    c_l = edge_label_index[1].astype(jnp.int32)
    N = x_user.shape[0]

    R1 = _rep(H1, C1, D1)
    R2 = _rep(H2, D1, H2 * D1)

    a1s_um = _ablk(l1_um_as, H1, C1)
    a1d_um = _ablk(l1_um_ad, H1, C1)
    a1s_mu = _ablk(l1_mu_as, H1, C1)
    a1d_mu = _ablk(l1_mu_ad, H1, C1)
    pad96 = jnp.zeros((128, 96), jnp.float32)
    wu1 = jnp.concatenate([l1_um_Ws, _mm(l1_um_Ws, a1s_um),
                           _mm(l1_mu_Wd, a1d_mu), pad96], axis=1)
    wm1 = jnp.concatenate([l1_mu_Ws, _mm(l1_um_Wd, a1d_um),
                           _mm(l1_mu_Ws, a1s_mu), pad96], axis=1)
    pu1 = _mm(x_user, wu1)
    pm1 = _mm(x_movie, wm1)

    g1_um, g1_mu = _sc_gather_pair(pu1, r, pm1, c)
    ex1_um, ex1_mu = _edge_ex(g1_um, g1_mu, D1)
    m1_um = _msg(g1_um, ex1_um, R1, 2)
    m1_mu = _msg(g1_mu, ex1_mu, R1, 2)
    den1_um, den1_mu = _sc_segsum_pair(ex1_um.reshape(1, -1, HP), c,
                                       ex1_mu.reshape(1, -1, HP), r, N)
    agg1_um, agg1_mu = _sc_segsum_pair(m1_um, c, m1_mu, r, N)
    zm = _finalize1(agg1_um, den1_um[0], R1, l1_um_b)
    zu = _finalize1(agg1_mu, den1_mu[0], R1, l1_mu_b)

    a2s_um = _ablk(l2_um_as, H2, C2)
    a2d_um = _ablk(l2_um_ad, H2, C2)
    a2s_mu = _ablk(l2_mu_as, H2, C2)
    a2d_mu = _ablk(l2_mu_ad, H2, C2)
    pad96b = jnp.zeros((D1, 96), jnp.float32)
    wu2 = jnp.concatenate([_mm(l2_um_Ws, a2s_um), _mm(l2_mu_Wd, a2d_mu),
                           pad96b], axis=1)
    wm2 = jnp.concatenate([_mm(l2_um_Wd, a2d_um), _mm(l2_mu_Ws, a2s_mu),
                           pad96b], axis=1)
    pu2 = jnp.concatenate([zu, _mm(zu, wu2)], axis=1)
    pm2 = jnp.concatenate([zm, _mm(zm, wm2)], axis=1)

    g2_um, g2_mu = _sc_gather_pair(pu2, r, pm2, c)
    ex2_um, ex2_mu = _edge_ex(g2_um, g2_mu, D1)
    m2_um = _msg(g2_um, ex2_um, R2, 2)
    m2_mu = _msg(g2_mu, ex2_mu, R2, 2)
    den2_um, den2_mu = _sc_segsum_pair(ex2_um.reshape(1, -1, HP), c,
                                       ex2_mu.reshape(1, -1, HP), r, N)
    agg2_um, agg2_mu = _sc_segsum_pair(m2_um, c, m2_mu, r, N)

    def _wp(Ws):
        blocks = []
        for j in range(H2 * D1 // WC):
            h, part = j // 2, j % 2
            blk = jnp.zeros((WC, D2), jnp.float32)
            blk = blk.at[:, h * C2:(h + 1) * C2].set(
                Ws[part * WC:(part + 1) * WC, h * C2:(h + 1) * C2])
            blocks.append(blk)
        return jnp.stack(blocks)

    zm2 = _finalize2(agg2_um, den2_um[0], R2, _wp(l2_um_Ws), l2_um_b)
    zu2 = _finalize2(agg2_mu, den2_mu[0], R2, _wp(l2_mu_Ws), l2_mu_b)

    w1pad = jnp.zeros((D2, 96), jnp.float32)
    pr = _mm(zu2, jnp.concatenate([dec_W1[:D2], w1pad], axis=1))
    pc = _mm(zm2, jnp.concatenate([dec_W1[D2:], w1pad], axis=1))
    dr, dc = _sc_gather_pair(pr, r_l, pc, c_l)
    return _decoder(dr, dc, dec_b1, dec_W2, dec_b2)
